# baseline (speedup 1.0000x reference)
import functools

import jax
import jax.numpy as jnp
from jax import lax
from jax.experimental import pallas as pl
from jax.experimental.pallas import tpu as pltpu
from jax.experimental.pallas import tpu_sc as plsc

B = 1024
H = 20
E = 8
CE = 64
PE = 64
OFFS = 64
NCAND = 4
DPFH = 3
DPF = DPFH * NCAND
NOUT = NCAND + 1 + OFFS
EO = E * OFFS
BB = 128
G = B // BB
R = H * BB
L = 16


def _t_body(c_ref, o_ref, ct_ref, ot_ref):
    cf = c_ref[...].astype(jnp.float32)
    v = lax.broadcasted_iota(jnp.int32, (B, B), 1)
    w = lax.broadcasted_iota(jnp.int32, (B, B), 0)
    src = (v // 128) * 128 + (v % 2) * 64 + (v % 128) // 2
    perm = (w == src).astype(jnp.float32)
    ct = lax.dot_general(cf, perm, (((0,), (0,)), ((), ())))
    ct_ref[...] = ct.astype(jnp.int32)
    ot_ref[...] = o_ref[...].T


def _transpose2(ch, oh):
    return pl.pallas_call(
        _t_body,
        out_shape=[
            jax.ShapeDtypeStruct((H, B), jnp.int32),
            jax.ShapeDtypeStruct((H, B), jnp.int32),
        ],
    )(ch, oh)


@functools.cache
def _sc_gather():
    info = plsc.get_sparse_core_info()
    nw = info.num_cores * info.num_subcores
    rc = (H * B) // nw
    rp = B // nw
    mesh = plsc.VectorSubcoreMesh(core_axis_name="c", subcore_axis_name="s")

    @functools.partial(
        pl.kernel,
        mesh=mesh,
        compiler_params=pltpu.CompilerParams(use_tc_tiling_on_sc=False),
        out_type=(
            jax.ShapeDtypeStruct((H * B, CE), jnp.float32),
        ),
        scratch_types=[
            pltpu.VMEM((rc,), jnp.int32),
            pltpu.VMEM((rc, CE), jnp.float32),
            pltpu.SemaphoreType.DMA,
        ],
    )
    def gather(ctab, cidx, cout, cidx_v, crows_v, sem):
        wid = lax.axis_index("s") * info.num_cores + lax.axis_index("c")
        cb = wid * rc
        pltpu.sync_copy(cidx.at[pl.ds(cb, rc)], cidx_v)
        pltpu.async_copy(ctab.at[cidx_v], crows_v, sem).wait()
        pltpu.sync_copy(crows_v, cout.at[pl.ds(cb, rc)])

    return gather


def _tc_body(ce_ref, oh_ref, ptab_ref, pc_ref, dpf_ref, off512_ref, offT_ref,
             wq_ref, wkT_ref, wv_ref, wo_ref,
             wpc_ref, wclctx_ref, wdpf_ref, b_ref,
             cand_ref, off_ref, pbuf_ref, psem):
    copies = []
    for i in range(BB):
        c = pltpu.make_async_copy(ptab_ref.at[pl.ds(pc_ref[i, 0], 1)],
                                  pbuf_ref.at[pl.ds(i, 1)], psem)
        c.start()
        copies.append(c)

    hb = BB // 2
    rh = H * hb
    pair = ce_ref[...]
    ce_lo = pair[:, :, :CE]
    ce_hi = pair[:, :, CE:]
    oh = oh_ref[...]

    qk = (wq_ref[...] @ (wkT_ref[...] @ offT_ref[...])) * 0.125
    vo = (off512_ref[...] @ wv_ref[...]) @ wo_ref[...]

    jcol = lax.broadcasted_iota(jnp.int32, (rh, EO), 1)
    ecol = lax.broadcasted_iota(jnp.int32, (EO, E), 0)
    p = (ecol % E == lax.broadcasted_iota(jnp.int32, (EO, E), 1)) \
        .astype(jnp.float32)
    zero = jnp.float32(0.0)

    sel_lo = (jcol // E) == oh[:, :hb, :].reshape(rh, 1)
    sel_hi = (jcol // E) == oh[:, hb:, :].reshape(rh, 1)
    s0_lo = jnp.dot(ce_lo.reshape(rh, CE), qk)
    s0_hi = jnp.dot(ce_hi.reshape(rh, CE), qk)
    sc0_lo = jnp.dot(jnp.where(sel_lo, s0_lo, zero), p)
    sc0_hi = jnp.dot(jnp.where(sel_hi, s0_hi, zero), p)

    for c in copies:
        c.wait()
    pce = pbuf_ref[...]
    s1b = jnp.dot(pce, qk)

    def _half(sel, sc0, s1h):
        s1 = jnp.broadcast_to(s1h[None], (H, hb, EO)).reshape(rh, EO)
        sc1 = jnp.dot(jnp.where(sel, s1, zero), p)
        ex = jnp.exp(jnp.concatenate([sc0, sc1], axis=1))
        e0, e1 = ex[:, :E], ex[:, E:]
        attn = 0.5 * (e0 / jnp.sum(e0, axis=1, keepdims=True)
                      + e1 / jnp.sum(e1, axis=1, keepdims=True))
        amat = jnp.where(sel, jnp.dot(attn, p.T), zero)
        return jnp.dot(amat, vo).reshape(H, hb, CE)

    ctx_lo = _half(sel_lo, sc0_lo, s1b[:hb])
    ctx_hi = _half(sel_hi, sc0_hi, s1b[hb:])

    base = dpf_ref[...] @ wdpf_ref[...] + b_ref[...] + pce @ wpc_ref[...]
    acc_lo = base[:hb]
    acc_hi = base[hb:]
    for h in range(H):
        both_lo = jnp.concatenate([ce_lo[h], ctx_lo[h]], axis=1)
        both_hi = jnp.concatenate([ce_hi[h], ctx_hi[h]], axis=1)
        acc_lo = acc_lo + both_lo @ wclctx_ref[h]
        acc_hi = acc_hi + both_hi @ wclctx_ref[h]
    cand_ref[:hb, :] = acc_lo[:, :NCAND + 1]
    cand_ref[hb:, :] = acc_hi[:, :NCAND + 1]
    off_ref[:hb, :] = acc_lo[:, NCAND + 1:]
    off_ref[hb:, :] = acc_hi[:, NCAND + 1:]


def _tc_call(ce3, oh3, pc_table, pc2, dpf2, off512, offT, Wq, WkT, Wv, Wo,
             Wpc, Wclctx, Wdpf, b2, interpret=False):
    full = lambda s: pl.BlockSpec(s, lambda j: (0,) * len(s))
    return pl.pallas_call(
        _tc_body,
        grid=(G,),
        in_specs=[
            pl.BlockSpec((H, BB // 2, 2 * CE), lambda j: (0, j, 0)),
            pl.BlockSpec((H, BB, 1), lambda j: (0, j, 0)),
            pl.BlockSpec(memory_space=pl.ANY),
            pl.BlockSpec((BB, 1), lambda j: (j, 0),
                         memory_space=pltpu.SMEM),
            pl.BlockSpec((BB, DPF), lambda j: (j, 0)),
            full((EO, CE)),
            full((CE, EO)),
            full((CE, CE)),
            full((CE, CE)),
            full((CE, CE)),
            full((CE, CE)),
            full((PE, NOUT)),
            full((H, 2 * CE, NOUT)),
            full((DPF, NOUT)),
            full((1, NOUT)),
        ],
        out_specs=[
            pl.BlockSpec((BB, NCAND + 1), lambda j: (j, 0)),
            pl.BlockSpec((BB, OFFS), lambda j: (j, 0)),
        ],
        out_shape=[
            jax.ShapeDtypeStruct((B, NCAND + 1), jnp.float32),
            jax.ShapeDtypeStruct((B, OFFS), jnp.float32),
        ],
        scratch_shapes=[
            pltpu.VMEM((BB, PE), jnp.float32),
            pltpu.SemaphoreType.DMA,
        ],
        interpret=interpret,
    )(ce3, oh3, pc_table, pc2, dpf2, off512, offT, Wq, WkT, Wv, Wo,
      Wpc, Wclctx, Wdpf, b2)


def kernel(cluster_history, offset_history, pc, dpf_vectors, pc_table,
           cluster_table, offset_table, Wq, Wk, Wv, Wo, W_cand, b_cand,
           W_off, b_off):
    chT, ohT = _transpose2(cluster_history, offset_history)
    v = jnp.arange(B)
    src = (v // 128) * 128 + (v % 2) * 64 + (v % 128) // 2
    cidx = cluster_history.T[:, src].reshape(-1)
    (ce_flat,) = _sc_gather()(cluster_table, cidx)
    ce3 = ce_flat.reshape(H, B // 2, 2 * CE)
    oh3 = ohT.reshape(H, B, 1)
    dpf2 = dpf_vectors.reshape(B, DPF)
    off512 = offset_table.reshape(EO, CE)
    offT = off512.T
    W = jnp.concatenate([W_cand, W_off], axis=1)
    b2 = jnp.concatenate([b_cand, b_off]).reshape(1, NOUT)
    Wpc = W[:PE]
    Wcl = W[PE:PE + H * CE].reshape(H, CE, NOUT)
    Wctx = W[PE + H * CE:PE + 2 * H * CE].reshape(H, CE, NOUT)
    Wclctx = jnp.concatenate([Wcl, Wctx], axis=1)
    Wdpf = W[PE + 2 * H * CE:]
    return tuple(_tc_call(ce3, oh3, pc_table, pc, dpf2, off512, offT, Wq,
                          Wk.T, Wv, Wo, Wpc, Wclctx, Wdpf, b2))


# speedup vs baseline: 1.3266x; 1.3266x over previous
import functools

import jax
import jax.numpy as jnp
from jax import lax
from jax.experimental import pallas as pl
from jax.experimental.pallas import tpu as pltpu
from jax.experimental.pallas import tpu_sc as plsc

B = 1024
H = 20
E = 8
CE = 64
PE = 64
OFFS = 64
NCAND = 4
DPFH = 3
DPF = DPFH * NCAND
NOUT = NCAND + 1 + OFFS
EO = E * OFFS
BB = 128
G = B // BB
R = H * BB
L = 16


def _t_body(c_ref, o_ref, ct_ref, ot_ref):
    cf = c_ref[...].astype(jnp.float32)
    v = lax.broadcasted_iota(jnp.int32, (B, B), 1)
    w = lax.broadcasted_iota(jnp.int32, (B, B), 0)
    src = (v // 128) * 128 + (v % 2) * 64 + (v % 128) // 2
    perm = (w == src).astype(jnp.float32)
    ct = lax.dot_general(cf, perm, (((0,), (0,)), ((), ())))
    ct_ref[...] = ct.astype(jnp.int32)
    ot_ref[...] = o_ref[...].T


def _transpose2(ch, oh):
    return pl.pallas_call(
        _t_body,
        out_shape=[
            jax.ShapeDtypeStruct((H, B), jnp.int32),
            jax.ShapeDtypeStruct((H, B), jnp.int32),
        ],
    )(ch, oh)


@functools.cache
def _sc_gather():
    info = plsc.get_sparse_core_info()
    nw = info.num_cores * info.num_subcores
    rc = (H * B) // nw
    rp = B // nw
    mesh = plsc.VectorSubcoreMesh(core_axis_name="c", subcore_axis_name="s")

    @functools.partial(
        pl.kernel,
        mesh=mesh,
        compiler_params=pltpu.CompilerParams(use_tc_tiling_on_sc=False),
        out_type=(
            jax.ShapeDtypeStruct((H * B, CE), jnp.float32),
        ),
        scratch_types=[
            pltpu.VMEM((rc,), jnp.int32),
            pltpu.VMEM((rc, CE), jnp.float32),
            pltpu.SemaphoreType.DMA,
        ],
    )
    def gather(ctab, cidx, cout, cidx_v, crows_v, sem):
        wid = lax.axis_index("s") * info.num_cores + lax.axis_index("c")
        cb = wid * rc
        pltpu.sync_copy(cidx.at[pl.ds(cb, rc)], cidx_v)
        pltpu.async_copy(ctab.at[cidx_v], crows_v, sem).wait()
        pltpu.sync_copy(crows_v, cout.at[pl.ds(cb, rc)])

    return gather


def _tc_body(ce_ref, oh_ref, ptab_ref, pc_ref, dpf_ref, off512_ref, offT_ref,
             wq_ref, wkT_ref, wv_ref, wo_ref,
             wpc_ref, wclctx_ref, wdpf_ref, b_ref,
             cand_ref, off_ref, pbuf_ref, psem):
    copies = []
    for i in range(BB):
        c = pltpu.make_async_copy(ptab_ref.at[pl.ds(pc_ref[i, 0], 1)],
                                  pbuf_ref.at[pl.ds(i, 1)], psem)
        c.start()
        copies.append(c)

    hb = BB // 2
    rh = H * hb
    pair = ce_ref[...]
    ce_lo = pair[:, :, :CE]
    ce_hi = pair[:, :, CE:]
    oh = oh_ref[...]

    qk = (wq_ref[...] @ (wkT_ref[...] @ offT_ref[...])) * 0.125
    vo = (off512_ref[...] @ wv_ref[...]) @ wo_ref[...]

    jcol = lax.broadcasted_iota(jnp.int32, (rh, EO), 1)
    ecol = lax.broadcasted_iota(jnp.int32, (EO, E), 0)
    p = (ecol % E == lax.broadcasted_iota(jnp.int32, (EO, E), 1)) \
        .astype(jnp.float32)
    zero = jnp.float32(0.0)

    sel_lo = (jcol // E) == oh[:, :hb, :].reshape(rh, 1)
    sel_hi = (jcol // E) == oh[:, hb:, :].reshape(rh, 1)
    s0_lo = jnp.dot(ce_lo.reshape(rh, CE), qk)
    s0_hi = jnp.dot(ce_hi.reshape(rh, CE), qk)
    sc0_lo = jnp.dot(jnp.where(sel_lo, s0_lo, zero), p)
    sc0_hi = jnp.dot(jnp.where(sel_hi, s0_hi, zero), p)

    for c in copies:
        c.wait()
    pce = pbuf_ref[...]
    s1b = jnp.dot(pce, qk)

    def _half(sel, sc0, s1h):
        s1 = jnp.broadcast_to(s1h[None], (H, hb, EO)).reshape(rh, EO)
        sc1 = jnp.dot(jnp.where(sel, s1, zero), p)
        attn = 0.5 * (jax.nn.softmax(sc0, axis=-1)
                      + jax.nn.softmax(sc1, axis=-1))
        amat = jnp.where(sel, jnp.dot(attn, p.T), zero)
        return jnp.dot(amat, vo).reshape(H, hb, CE)

    ctx_lo = _half(sel_lo, sc0_lo, s1b[:hb])
    ctx_hi = _half(sel_hi, sc0_hi, s1b[hb:])

    base = dpf_ref[...] @ wdpf_ref[...] + b_ref[...] + pce @ wpc_ref[...]
    acc_lo = base[:hb]
    acc_hi = base[hb:]
    for h in range(H):
        acc_lo = (acc_lo + ce_lo[h] @ wclctx_ref[h, :CE]
                  + ctx_lo[h] @ wclctx_ref[h, CE:])
        acc_hi = (acc_hi + ce_hi[h] @ wclctx_ref[h, :CE]
                  + ctx_hi[h] @ wclctx_ref[h, CE:])
    cand_ref[:hb, :] = acc_lo[:, :NCAND + 1]
    cand_ref[hb:, :] = acc_hi[:, :NCAND + 1]
    off_ref[:hb, :] = acc_lo[:, NCAND + 1:]
    off_ref[hb:, :] = acc_hi[:, NCAND + 1:]


def _tc_call(ce3, oh3, pc_table, pc2, dpf2, off512, offT, Wq, WkT, Wv, Wo,
             Wpc, Wclctx, Wdpf, b2, interpret=False):
    full = lambda s: pl.BlockSpec(s, lambda j: (0,) * len(s))
    return pl.pallas_call(
        _tc_body,
        grid=(G,),
        in_specs=[
            pl.BlockSpec((H, BB // 2, 2 * CE), lambda j: (0, j, 0)),
            pl.BlockSpec((H, BB, 1), lambda j: (0, j, 0)),
            pl.BlockSpec(memory_space=pl.ANY),
            pl.BlockSpec((BB, 1), lambda j: (j, 0),
                         memory_space=pltpu.SMEM),
            pl.BlockSpec((BB, DPF), lambda j: (j, 0)),
            full((EO, CE)),
            full((CE, EO)),
            full((CE, CE)),
            full((CE, CE)),
            full((CE, CE)),
            full((CE, CE)),
            full((PE, NOUT)),
            full((H, 2 * CE, NOUT)),
            full((DPF, NOUT)),
            full((1, NOUT)),
        ],
        out_specs=[
            pl.BlockSpec((BB, NCAND + 1), lambda j: (j, 0)),
            pl.BlockSpec((BB, OFFS), lambda j: (j, 0)),
        ],
        out_shape=[
            jax.ShapeDtypeStruct((B, NCAND + 1), jnp.float32),
            jax.ShapeDtypeStruct((B, OFFS), jnp.float32),
        ],
        scratch_shapes=[
            pltpu.VMEM((BB, PE), jnp.float32),
            pltpu.SemaphoreType.DMA,
        ],
        interpret=interpret,
    )(ce3, oh3, pc_table, pc2, dpf2, off512, offT, Wq, WkT, Wv, Wo,
      Wpc, Wclctx, Wdpf, b2)


def kernel(cluster_history, offset_history, pc, dpf_vectors, pc_table,
           cluster_table, offset_table, Wq, Wk, Wv, Wo, W_cand, b_cand,
           W_off, b_off):
    chT, ohT = _transpose2(cluster_history, offset_history)
    v = jnp.arange(B)
    src = (v // 128) * 128 + (v % 2) * 64 + (v % 128) // 2
    cidx = cluster_history.T[:, src].reshape(-1)
    (ce_flat,) = _sc_gather()(cluster_table, cidx)
    ce3 = ce_flat.reshape(H, B // 2, 2 * CE)
    oh3 = ohT.reshape(H, B, 1)
    dpf2 = dpf_vectors.reshape(B, DPF)
    off512 = offset_table.reshape(EO, CE)
    offT = off512.T
    W = jnp.concatenate([W_cand, W_off], axis=1)
    b2 = jnp.concatenate([b_cand, b_off]).reshape(1, NOUT)
    Wpc = W[:PE]
    Wcl = W[PE:PE + H * CE].reshape(H, CE, NOUT)
    Wctx = W[PE + H * CE:PE + 2 * H * CE].reshape(H, CE, NOUT)
    Wclctx = jnp.concatenate([Wcl, Wctx], axis=1)
    Wdpf = W[PE + 2 * H * CE:]
    return tuple(_tc_call(ce3, oh3, pc_table, pc, dpf2, off512, offT, Wq,
                          Wk.T, Wv, Wo, Wpc, Wclctx, Wdpf, b2))


# speedup vs baseline: 1.3275x; 1.0006x over previous
import functools

import jax
import jax.numpy as jnp
from jax import lax
from jax.experimental import pallas as pl
from jax.experimental.pallas import tpu as pltpu
from jax.experimental.pallas import tpu_sc as plsc

B = 1024
H = 20
E = 8
CE = 64
PE = 64
OFFS = 64
NCAND = 4
DPFH = 3
DPF = DPFH * NCAND
NOUT = NCAND + 1 + OFFS
EO = E * OFFS
BB = 128
G = B // BB
R = H * BB


def _t_body(c_ref, o_ref, ct_ref, ot_ref):
    cf = c_ref[...].astype(jnp.float32)
    v = lax.broadcasted_iota(jnp.int32, (B, B), 1)
    w = lax.broadcasted_iota(jnp.int32, (B, B), 0)
    src = (v // 128) * 128 + (v % 2) * 64 + (v % 128) // 2
    perm = (w == src).astype(jnp.float32)
    ct = lax.dot_general(cf, perm, (((0,), (0,)), ((), ())))
    ct_ref[...] = ct.astype(jnp.int32)
    ot_ref[...] = o_ref[...].T


def _transpose2(ch, oh):
    return pl.pallas_call(
        _t_body,
        out_shape=[
            jax.ShapeDtypeStruct((H, B), jnp.int32),
            jax.ShapeDtypeStruct((H, B), jnp.int32),
        ],
    )(ch, oh)


@functools.cache
def _sc_gather():
    info = plsc.get_sparse_core_info()
    nw = info.num_cores * info.num_subcores
    rc = (H * B) // nw
    rp = B // nw
    mesh = plsc.VectorSubcoreMesh(core_axis_name="c", subcore_axis_name="s")

    @functools.partial(
        pl.kernel,
        mesh=mesh,
        compiler_params=pltpu.CompilerParams(use_tc_tiling_on_sc=False),
        out_type=(
            jax.ShapeDtypeStruct((H * B, CE), jnp.float32),
        ),
        scratch_types=[
            pltpu.VMEM((rc,), jnp.int32),
            pltpu.VMEM((rc, CE), jnp.float32),
            pltpu.SemaphoreType.DMA,
        ],
    )
    def gather(ctab, cidx, cout, cidx_v, crows_v, sem):
        wid = lax.axis_index("s") * info.num_cores + lax.axis_index("c")
        cb = wid * rc
        pltpu.sync_copy(cidx.at[pl.ds(cb, rc)], cidx_v)
        pltpu.async_copy(ctab.at[cidx_v], crows_v, sem).wait()
        pltpu.sync_copy(crows_v, cout.at[pl.ds(cb, rc)])

    return gather


def _tc_body(ce_ref, oh_ref, ptab_ref, pc_ref, dpf_ref, off512_ref, offT_ref,
             wq_ref, wkT_ref, wv_ref, wo_ref,
             wpc_ref, wclctx_ref, wdpf_ref, b_ref,
             cand_ref, off_ref, pbuf_ref, psem):
    copies = []
    for i in range(BB):
        c = pltpu.make_async_copy(ptab_ref.at[pl.ds(pc_ref[i, 0], 1)],
                                  pbuf_ref.at[pl.ds(i, 1)], psem)
        c.start()
        copies.append(c)

    hb = BB // 2
    rh = H * hb
    pair = ce_ref[...]
    ce_lo = pair[:, :, :CE]
    ce_hi = pair[:, :, CE:]
    oh = oh_ref[...]

    qk = (wq_ref[...] @ (wkT_ref[...] @ offT_ref[...])) * 0.125
    vo = (off512_ref[...] @ wv_ref[...]) @ wo_ref[...]

    jcol = lax.broadcasted_iota(jnp.int32, (rh, EO), 1)
    ecol = lax.broadcasted_iota(jnp.int32, (EO, E), 0)
    p = (ecol % E == lax.broadcasted_iota(jnp.int32, (EO, E), 1)) \
        .astype(jnp.float32)
    zero = jnp.float32(0.0)

    sel_lo = (jcol // E) == oh[:, :hb, :].reshape(rh, 1)
    sel_hi = (jcol // E) == oh[:, hb:, :].reshape(rh, 1)
    s0_lo = jnp.dot(ce_lo.reshape(rh, CE), qk)
    s0_hi = jnp.dot(ce_hi.reshape(rh, CE), qk)
    sc0_lo = jnp.dot(jnp.where(sel_lo, s0_lo, zero), p)
    sc0_hi = jnp.dot(jnp.where(sel_hi, s0_hi, zero), p)

    for c in copies:
        c.wait()
    pce = pbuf_ref[...]
    s1b = jnp.dot(pce, qk)

    def _half(sel, sc0, s1h):
        s1 = jnp.broadcast_to(s1h[None], (H, hb, EO)).reshape(rh, EO)
        sc1 = jnp.dot(jnp.where(sel, s1, zero), p)
        attn = 0.5 * (jax.nn.softmax(sc0, axis=-1)
                      + jax.nn.softmax(sc1, axis=-1))
        amat = jnp.where(sel, jnp.dot(attn, p.T), zero)
        return jnp.dot(amat, vo).reshape(H, hb, CE)

    ctx_lo = _half(sel_lo, sc0_lo, s1b[:hb])
    ctx_hi = _half(sel_hi, sc0_hi, s1b[hb:])

    base = dpf_ref[...] @ wdpf_ref[...] + b_ref[...] + pce @ wpc_ref[...]
    acc_lo = base[:hb]
    acc_hi = base[hb:]
    for h in range(H):
        acc_lo = (acc_lo + ce_lo[h] @ wclctx_ref[h, :CE]
                  + ctx_lo[h] @ wclctx_ref[h, CE:])
        acc_hi = (acc_hi + ce_hi[h] @ wclctx_ref[h, :CE]
                  + ctx_hi[h] @ wclctx_ref[h, CE:])
    cand_ref[:hb, :] = acc_lo[:, :NCAND + 1]
    cand_ref[hb:, :] = acc_hi[:, :NCAND + 1]
    off_ref[:hb, :] = acc_lo[:, NCAND + 1:]
    off_ref[hb:, :] = acc_hi[:, NCAND + 1:]


def _tc_call(ce3, oh3, pc_table, pc2, dpf2, off512, offT, Wq, WkT, Wv, Wo,
             Wpc, Wclctx, Wdpf, b2, interpret=False):
    full = lambda s: pl.BlockSpec(s, lambda j: (0,) * len(s))
    return pl.pallas_call(
        _tc_body,
        grid=(G,),
        in_specs=[
            pl.BlockSpec((H, BB // 2, 2 * CE), lambda j: (0, j, 0)),
            pl.BlockSpec((H, BB, 1), lambda j: (0, j, 0)),
            pl.BlockSpec(memory_space=pl.ANY),
            pl.BlockSpec((BB, 1), lambda j: (j, 0),
                         memory_space=pltpu.SMEM),
            pl.BlockSpec((BB, DPF), lambda j: (j, 0)),
            full((EO, CE)),
            full((CE, EO)),
            full((CE, CE)),
            full((CE, CE)),
            full((CE, CE)),
            full((CE, CE)),
            full((PE, NOUT)),
            full((H, 2 * CE, NOUT)),
            full((DPF, NOUT)),
            full((1, NOUT)),
        ],
        out_specs=[
            pl.BlockSpec((BB, NCAND + 1), lambda j: (j, 0)),
            pl.BlockSpec((BB, OFFS), lambda j: (j, 0)),
        ],
        out_shape=[
            jax.ShapeDtypeStruct((B, NCAND + 1), jnp.float32),
            jax.ShapeDtypeStruct((B, OFFS), jnp.float32),
        ],
        scratch_shapes=[
            pltpu.VMEM((BB, PE), jnp.float32),
            pltpu.SemaphoreType.DMA,
        ],
        interpret=interpret,
    )(ce3, oh3, pc_table, pc2, dpf2, off512, offT, Wq, WkT, Wv, Wo,
      Wpc, Wclctx, Wdpf, b2)


def kernel(cluster_history, offset_history, pc, dpf_vectors, pc_table,
           cluster_table, offset_table, Wq, Wk, Wv, Wo, W_cand, b_cand,
           W_off, b_off):
    chT, ohT = _transpose2(cluster_history, offset_history)
    v = jnp.arange(B)
    src = (v // 128) * 128 + (v % 2) * 64 + (v % 128) // 2
    cidx = cluster_history.T[:, src].reshape(-1)
    (ce_flat,) = _sc_gather()(cluster_table, cidx)
    ce3 = ce_flat.reshape(H, B // 2, 2 * CE)
    oh3 = ohT.reshape(H, B, 1)
    dpf2 = dpf_vectors.reshape(B, DPF)
    off512 = offset_table.reshape(EO, CE)
    offT = off512.T
    W = jnp.concatenate([W_cand, W_off], axis=1)
    b2 = jnp.concatenate([b_cand, b_off]).reshape(1, NOUT)
    Wpc = W[:PE]
    Wcl = W[PE:PE + H * CE].reshape(H, CE, NOUT)
    Wctx = W[PE + H * CE:PE + 2 * H * CE].reshape(H, CE, NOUT)
    Wclctx = jnp.concatenate([Wcl, Wctx], axis=1)
    Wdpf = W[PE + 2 * H * CE:]
    return tuple(_tc_call(ce3, oh3, pc_table, pc, dpf2, off512, offT, Wq,
                          Wk.T, Wv, Wo, Wpc, Wclctx, Wdpf, b2))
